# baseline (speedup 1.0000x reference)
import functools

import jax
import jax.numpy as jnp
from jax import lax
from jax.experimental import pallas as pl
from jax.experimental.pallas import tpu as pltpu
from jax.experimental.pallas import tpu_sc as plsc

_H = 224
_W = 224
_F = 4
_DTOT = 384
_PLANES = _F * _DTOT
_LANES = 16
_WJ = _W // _LANES
_HH = _H // 2
_NWORKERS = 32
_PER_W = _PLANES // _NWORKERS


def _materialize_planes(u, v):
    mesh = plsc.VectorSubcoreMesh(core_axis_name="c", subcore_axis_name="s")

    @functools.partial(
        pl.kernel,
        mesh=mesh,
        out_type=jax.ShapeDtypeStruct((_F, _DTOT, _H, _W), jnp.float32),
        scratch_types=[
            pltpu.VMEM((_HH, _W), jnp.float32),
            pltpu.VMEM((_HH, _W), jnp.float32),
            pltpu.VMEM((_PER_W, _H), jnp.float32),
            pltpu.VMEM((_PER_W, _W), jnp.float32),
            pltpu.SemaphoreType.DMA,
            pltpu.SemaphoreType.DMA,
        ],
        compiler_params=pltpu.CompilerParams(use_tc_tiling_on_sc=True),
    )
    def kern(u_hbm, v_hbm, out_hbm, plane_a, plane_b, ubuf, vbuf, sem_a, sem_b):
        wid = lax.axis_index("s") * 2 + lax.axis_index("c")
        base = wid * _PER_W
        kk = wid // (_DTOT // _PER_W)
        dbase = (wid % (_DTOT // _PER_W)) * _PER_W

        pltpu.sync_copy(u_hbm.at[pl.ds(base, _PER_W)], ubuf)
        pltpu.sync_copy(v_hbm.at[pl.ds(base, _PER_W)], vbuf)

        def build_half(plane, lp, half):
            vvecs = [vbuf[lp, pl.ds(jj * _LANES, _LANES)] for jj in range(_WJ)]

            def grp(ig, carry):
                uv = ubuf[lp, pl.ds(half * _HH + ig * _LANES, _LANES)]
                for lane in range(_LANES):
                    i = ig * _LANES + lane
                    sv = jnp.full((_LANES,), uv[lane], dtype=jnp.float32)
                    for jj in range(_WJ):
                        plane[i, pl.ds(jj * _LANES, _LANES)] = sv * vvecs[jj]
                return carry

            lax.fori_loop(0, _HH // _LANES, grp, 0)

        def step(p, carry):

            @pl.when(p > 0)
            def _():
                pltpu.make_async_copy(
                    plane_a, out_hbm.at[kk, dbase + p, pl.ds(0, _HH)], sem_a).wait()

            build_half(plane_a, p, 0)
            pltpu.make_async_copy(
                plane_a, out_hbm.at[kk, dbase + p, pl.ds(0, _HH)], sem_a).start()

            @pl.when(p > 0)
            def _():
                pltpu.make_async_copy(
                    plane_b, out_hbm.at[kk, dbase + p, pl.ds(_HH, _HH)], sem_b).wait()

            build_half(plane_b, p, 1)
            pltpu.make_async_copy(
                plane_b, out_hbm.at[kk, dbase + p, pl.ds(_HH, _HH)], sem_b).start()
            return carry

        lax.fori_loop(0, _PER_W, step, 0)

        pltpu.make_async_copy(plane_a, out_hbm.at[kk, dbase, pl.ds(0, _HH)], sem_a).wait()
        pltpu.make_async_copy(plane_b, out_hbm.at[kk, dbase, pl.ds(_HH, _HH)], sem_b).wait()

    return kern(u, v)


def kernel(patch, num_views, row_weight, col_weight, frame_weight):
    h, w = patch.shape[2], patch.shape[3]
    f = _F
    cw = col_weight[:h]
    rw = row_weight[:w]
    fw = frame_weight[:f]
    d0, d1, d2 = cw.shape[1], rw.shape[1], fw.shape[1]

    ua = jnp.broadcast_to(cw.T[None], (f, d0, h))
    ub = jnp.ones((f, d1, h), jnp.float32)
    uc = jnp.broadcast_to(fw[:, :, None], (f, d2, h))
    u = jnp.concatenate([ua, ub, uc], axis=1).reshape(f * _DTOT, h)

    va = jnp.ones((f, d0, w), jnp.float32)
    vb = jnp.broadcast_to(rw.T[None], (f, d1, w))
    vc = jnp.ones((f, d2, w), jnp.float32)
    v = jnp.concatenate([va, vb, vc], axis=1).reshape(f * _DTOT, w)

    return _materialize_planes(u, v)


# speedup vs baseline: 1.1663x; 1.1663x over previous
import functools

import jax
import jax.numpy as jnp
from jax import lax
from jax.experimental import pallas as pl
from jax.experimental.pallas import tpu as pltpu
from jax.experimental.pallas import tpu_sc as plsc

_H = 224
_W = 224
_F = 4
_DTOT = 384
_PLANES = _F * _DTOT
_LANES = 16
_WJ = _W // _LANES
_HH = _H // 2
_NWORKERS = 32
_PER_W = _PLANES // _NWORKERS


def _materialize_planes(u, v):
    mesh = plsc.VectorSubcoreMesh(core_axis_name="c", subcore_axis_name="s")

    @functools.partial(
        pl.kernel,
        mesh=mesh,
        out_type=jax.ShapeDtypeStruct((_PLANES, _H, _W), jnp.float32),
        scratch_types=[
            pltpu.VMEM((_HH, _W), jnp.float32),
            pltpu.VMEM((_HH, _W), jnp.float32),
            pltpu.VMEM((_PER_W, _H), jnp.float32),
            pltpu.VMEM((_PER_W, _W), jnp.float32),
            pltpu.SemaphoreType.DMA,
            pltpu.SemaphoreType.DMA,
        ],
        compiler_params=pltpu.CompilerParams(use_tc_tiling_on_sc=True, needs_layout_passes=True),
    )
    def kern(u_hbm, v_hbm, out_hbm, plane_a, plane_b, ubuf, vbuf, sem_a, sem_b):
        wid = lax.axis_index("s") * 2 + lax.axis_index("c")
        base = wid * _PER_W

        pltpu.sync_copy(u_hbm.at[pl.ds(base, _PER_W)], ubuf)
        pltpu.sync_copy(v_hbm.at[pl.ds(base, _PER_W)], vbuf)

        def build_half(plane, lp, half):
            vvecs = [vbuf[lp, pl.ds(jj * _LANES, _LANES)] for jj in range(_WJ)]

            def grp(ig, carry):
                uv = ubuf[lp, pl.ds(half * _HH + ig * _LANES, _LANES)]
                for lane in range(_LANES):
                    i = ig * _LANES + lane
                    sv = jnp.full((_LANES,), uv[lane], dtype=jnp.float32)
                    for jj in range(_WJ):
                        plane[i, pl.ds(jj * _LANES, _LANES)] = sv * vvecs[jj]
                return carry

            lax.fori_loop(0, _HH // _LANES, grp, 0)

        def step(p, carry):

            @pl.when(p > 0)
            def _():
                pltpu.make_async_copy(
                    plane_a, out_hbm.at[base + p, pl.ds(0, _HH)], sem_a).wait()

            build_half(plane_a, p, 0)
            pltpu.make_async_copy(
                plane_a, out_hbm.at[base + p, pl.ds(0, _HH)], sem_a).start()

            @pl.when(p > 0)
            def _():
                pltpu.make_async_copy(
                    plane_b, out_hbm.at[base + p, pl.ds(_HH, _HH)], sem_b).wait()

            build_half(plane_b, p, 1)
            pltpu.make_async_copy(
                plane_b, out_hbm.at[base + p, pl.ds(_HH, _HH)], sem_b).start()
            return carry

        lax.fori_loop(0, _PER_W, step, 0)

        pltpu.make_async_copy(plane_a, out_hbm.at[base, pl.ds(0, _HH)], sem_a).wait()
        pltpu.make_async_copy(plane_b, out_hbm.at[base, pl.ds(_HH, _HH)], sem_b).wait()

    return kern(u, v)


def kernel(patch, num_views, row_weight, col_weight, frame_weight):
    h, w = patch.shape[2], patch.shape[3]
    f = _F
    cw = col_weight[:h]
    rw = row_weight[:w]
    fw = frame_weight[:f]
    d0, d1, d2 = cw.shape[1], rw.shape[1], fw.shape[1]

    ua = jnp.broadcast_to(cw.T[None], (f, d0, h))
    ub = jnp.ones((f, d1, h), jnp.float32)
    uc = jnp.broadcast_to(fw[:, :, None], (f, d2, h))
    u = jnp.concatenate([ua, ub, uc], axis=1).reshape(f * _DTOT, h)

    va = jnp.ones((f, d0, w), jnp.float32)
    vb = jnp.broadcast_to(rw.T[None], (f, d1, w))
    vc = jnp.ones((f, d2, w), jnp.float32)
    v = jnp.concatenate([va, vb, vc], axis=1).reshape(f * _DTOT, w)

    return _materialize_planes(u, v).reshape(f, _DTOT, h, w)


# speedup vs baseline: 3.8985x; 3.3427x over previous
import functools

import jax
import jax.numpy as jnp
from jax import lax
from jax.experimental import pallas as pl
from jax.experimental.pallas import tpu as pltpu
from jax.experimental.pallas import tpu_sc as plsc

_H = 224
_W = 224
_HW = _W // 2
_F = 4
_DSUB = 128
_LANES = 16
_VJ = _DSUB // _LANES
_NWORKERS = 32
_SLABS = _F * _H
_PER_W = _SLABS // _NWORKERS


def _materialize(cw, rw, fw):
    mesh = plsc.VectorSubcoreMesh(core_axis_name="c", subcore_axis_name="s")

    @functools.partial(
        pl.kernel,
        mesh=mesh,
        out_type=jax.ShapeDtypeStruct((_F, _H, _W, 3 * _DSUB), jnp.float32),
        scratch_types=[
            pltpu.VMEM((_HW, _DSUB), jnp.float32),
            pltpu.VMEM((_HW, _DSUB), jnp.float32),
            pltpu.VMEM((_W, _DSUB), jnp.float32),
            pltpu.VMEM((_HW, _DSUB), jnp.float32),
            pltpu.VMEM((_H, _DSUB), jnp.float32),
            pltpu.VMEM((_F, _DSUB), jnp.float32),
            pltpu.SemaphoreType.DMA,
            pltpu.SemaphoreType.DMA,
            pltpu.SemaphoreType.DMA,
        ],
    )
    def kern(cw_hbm, rw_hbm, fw_hbm, x_hbm, cbuf_a, cbuf_b, rbuf, fbuf,
             cstage, fstage, sem_a, sem_b, sem_rf):
        wid = lax.axis_index("s") * 2 + lax.axis_index("c")
        per_k = _H // _PER_W
        kk = wid // per_k
        ibase = (wid % per_k) * _PER_W

        pltpu.sync_copy(rw_hbm, rbuf)
        pltpu.sync_copy(cw_hbm, cstage)
        pltpu.sync_copy(fw_hbm, fstage)

        def fill(buf, vecs):
            def row(j, carry):
                for m in range(_VJ):
                    buf[j, pl.ds(m * _LANES, _LANES)] = vecs[m]
                return carry

            lax.fori_loop(0, _HW, row, 0)

        def start2(buf, ii, dlo, sem):
            for jlo in (0, _HW):
                pltpu.make_async_copy(
                    buf, x_hbm.at[kk, ii, pl.ds(jlo, _HW), pl.ds(dlo, _DSUB)],
                    sem).start()

        def wait2(buf, dlo, sem):
            for jlo in (0, _HW):
                pltpu.make_async_copy(
                    buf, x_hbm.at[kk, ibase, pl.ds(jlo, _HW),
                                  pl.ds(dlo, _DSUB)], sem).wait()

        fill(fbuf, [fstage[kk, pl.ds(m * _LANES, _LANES)] for m in range(_VJ)])

        def step(p, carry):
            ii = ibase + p

            @pl.when(p % 2 == 0)
            def _():
                @pl.when(p >= 2)
                def _():
                    wait2(cbuf_a, 0, sem_a)

                fill(cbuf_a, [cstage[ii, pl.ds(m * _LANES, _LANES)]
                              for m in range(_VJ)])
                start2(cbuf_a, ii, 0, sem_a)

            @pl.when(p % 2 == 1)
            def _():
                @pl.when(p >= 3)
                def _():
                    wait2(cbuf_b, 0, sem_b)

                fill(cbuf_b, [cstage[ii, pl.ds(m * _LANES, _LANES)]
                              for m in range(_VJ)])
                start2(cbuf_b, ii, 0, sem_b)

            pltpu.make_async_copy(
                rbuf, x_hbm.at[kk, ii, :, pl.ds(_DSUB, _DSUB)], sem_rf).start()
            start2(fbuf, ii, 2 * _DSUB, sem_rf)
            return carry

        lax.fori_loop(0, _PER_W, step, 0)

        wait2(cbuf_a, 0, sem_a)
        wait2(cbuf_b, 0, sem_b)

        def drain(p, carry):
            pltpu.make_async_copy(
                rbuf, x_hbm.at[kk, ibase, :, pl.ds(_DSUB, _DSUB)],
                sem_rf).wait()
            wait2(fbuf, 2 * _DSUB, sem_rf)
            return carry

        lax.fori_loop(0, _PER_W, drain, 0)

    return kern(cw, rw, fw)


def kernel(patch, num_views, row_weight, col_weight, frame_weight):
    h, w = patch.shape[2], patch.shape[3]
    cw = col_weight[:h]
    rw = row_weight[:w]
    fw = frame_weight[:_F]
    x = _materialize(cw, rw, fw)
    return jnp.transpose(x, (0, 3, 1, 2))


# speedup vs baseline: 3.9390x; 1.0104x over previous
import functools

import jax
import jax.numpy as jnp
from jax import lax
from jax.experimental import pallas as pl
from jax.experimental.pallas import tpu as pltpu
from jax.experimental.pallas import tpu_sc as plsc

_H = 224
_W = 224
_HW = _W // 2
_F = 4
_DSUB = 128
_LANES = 16
_VJ = _DSUB // _LANES
_NWORKERS = 32
_SLABS = _F * _H
_PER_W = _SLABS // _NWORKERS


def _materialize(cw, rw, fw):
    mesh = plsc.VectorSubcoreMesh(core_axis_name="c", subcore_axis_name="s")

    @functools.partial(
        pl.kernel,
        mesh=mesh,
        out_type=jax.ShapeDtypeStruct((_F, _H, _W, 3 * _DSUB), jnp.float32),
        scratch_types=[
            pltpu.VMEM((_HW, _DSUB), jnp.float32),
            pltpu.VMEM((_HW, _DSUB), jnp.float32),
            pltpu.VMEM((_W, _DSUB), jnp.float32),
            pltpu.VMEM((_HW, _DSUB), jnp.float32),
            pltpu.VMEM((_H, _DSUB), jnp.float32),
            pltpu.VMEM((_F, _DSUB), jnp.float32),
            pltpu.SemaphoreType.DMA,
            pltpu.SemaphoreType.DMA,
            pltpu.SemaphoreType.DMA,
        ],
    )
    def kern(cw_hbm, rw_hbm, fw_hbm, x_hbm, cbuf_a, cbuf_b, rbuf, fbuf,
             cstage, fstage, sem_a, sem_b, sem_rf):
        wid = lax.axis_index("s") * 2 + lax.axis_index("c")
        per_k = _H // _PER_W
        kk = wid // per_k
        ibase = (wid % per_k) * _PER_W

        pltpu.sync_copy(rw_hbm.at[pl.ds(0, _W)], rbuf)
        pltpu.sync_copy(cw_hbm.at[pl.ds(0, _H)], cstage)
        pltpu.sync_copy(fw_hbm.at[pl.ds(0, _F)], fstage)

        def fill(buf, vecs):
            def row(j, carry):
                for m in range(_VJ):
                    buf[j, pl.ds(m * _LANES, _LANES)] = vecs[m]
                return carry

            lax.fori_loop(0, _HW, row, 0)

        def start2(buf, ii, dlo, sem):
            for jlo in (0, _HW):
                pltpu.make_async_copy(
                    buf, x_hbm.at[kk, ii, pl.ds(jlo, _HW), pl.ds(dlo, _DSUB)],
                    sem).start()

        def wait2(buf, dlo, sem):
            for jlo in (0, _HW):
                pltpu.make_async_copy(
                    buf, x_hbm.at[kk, ibase, pl.ds(jlo, _HW),
                                  pl.ds(dlo, _DSUB)], sem).wait()

        fill(fbuf, [fstage[kk, pl.ds(m * _LANES, _LANES)] for m in range(_VJ)])

        def step(p, carry):
            ii = ibase + p

            @pl.when(p % 2 == 0)
            def _():
                @pl.when(p >= 2)
                def _():
                    wait2(cbuf_a, 0, sem_a)

                fill(cbuf_a, [cstage[ii, pl.ds(m * _LANES, _LANES)]
                              for m in range(_VJ)])
                start2(cbuf_a, ii, 0, sem_a)

            @pl.when(p % 2 == 1)
            def _():
                @pl.when(p >= 3)
                def _():
                    wait2(cbuf_b, 0, sem_b)

                fill(cbuf_b, [cstage[ii, pl.ds(m * _LANES, _LANES)]
                              for m in range(_VJ)])
                start2(cbuf_b, ii, 0, sem_b)

            pltpu.make_async_copy(
                rbuf, x_hbm.at[kk, ii, :, pl.ds(_DSUB, _DSUB)], sem_rf).start()
            start2(fbuf, ii, 2 * _DSUB, sem_rf)
            return carry

        lax.fori_loop(0, _PER_W, step, 0)

        wait2(cbuf_a, 0, sem_a)
        wait2(cbuf_b, 0, sem_b)

        def drain(p, carry):
            pltpu.make_async_copy(
                rbuf, x_hbm.at[kk, ibase, :, pl.ds(_DSUB, _DSUB)],
                sem_rf).wait()
            wait2(fbuf, 2 * _DSUB, sem_rf)
            return carry

        lax.fori_loop(0, _PER_W, drain, 0)

    return kern(cw, rw, fw)


def kernel(patch, num_views, row_weight, col_weight, frame_weight):
    x = _materialize(col_weight, row_weight, frame_weight)
    return jnp.transpose(x, (0, 3, 1, 2))
